# baseline (speedup 1.0000x reference)
import functools

import jax
import jax.numpy as jnp
from jax import lax
from jax.experimental import pallas as pl
from jax.experimental.pallas import tpu as pltpu
from jax.experimental.pallas import tpu_sc as plsc

N = 2048
KNN = 20
F = 64
NEG_BIG = -1e30


def _knn_body(x_ref, xt_ref, idx_ref, *, blk, n, k):
    b = pl.program_id(0)
    xb = x_ref[0]
    xt = xt_ref[0]
    inner = jnp.dot(xb.astype(jnp.bfloat16), xt.astype(jnp.bfloat16),
                    preferred_element_type=jnp.float32)
    sq_row = jnp.sum(xt * xt, axis=0, keepdims=True)
    sq_i = jnp.sum(xb * xb, axis=1, keepdims=True)
    neg = -(sq_i - 2.0 * inner + sq_row)
    lane = lax.broadcasted_iota(jnp.int32, (blk, n), 1)
    cols = []
    for _ in range(k):
        j = jnp.argmax(neg, axis=1).astype(jnp.int32)[:, None]
        cols.append(j)
        neg = jnp.where(lane == j, NEG_BIG, neg)
    idx_ref[0] = jnp.concatenate(cols, axis=1) + b * n


def _knn_topk(x):
    bb, n, c = x.shape
    blk = 512
    xt = jnp.swapaxes(x, 1, 2)
    return pl.pallas_call(
        functools.partial(_knn_body, blk=blk, n=n, k=KNN),
        grid=(bb, n // blk),
        in_specs=[
            pl.BlockSpec((1, blk, c), lambda b, i: (b, i, 0)),
            pl.BlockSpec((1, c, n), lambda b, i: (b, 0, 0)),
        ],
        out_specs=pl.BlockSpec((1, blk, KNN), lambda b, i: (b, i, 0)),
        out_shape=jax.ShapeDtypeStruct((bb, n, KNN), jnp.int32),
    )(x, xt)


_CHUNK = 128


def _sc_gather(table, idx):
    m = idx.shape[0]
    width = table.shape[1]
    nw = 32
    rows_per_w = m // nw
    n_chunks = rows_per_w // _CHUNK
    mesh = plsc.VectorSubcoreMesh(core_axis_name="c", subcore_axis_name="s")

    nbuf = 4
    assert n_chunks % nbuf == 0

    def body(table_hbm, idx_hbm, out_hbm, idx_v, rows_v, sem):
        wid = lax.axis_index("s") * 2 + lax.axis_index("c")
        base = wid * rows_per_w

        def step(g, carry):
            off0 = base + g * (nbuf * _CHUNK)
            handles = []
            for v in range(nbuf):
                off = off0 + v * _CHUNK
                pltpu.sync_copy(idx_hbm.at[pl.ds(off, _CHUNK)], idx_v.at[v])
                handles.append(
                    pltpu.async_copy(table_hbm.at[idx_v.at[v]], rows_v.at[v],
                                     sem))
            for h in handles:
                h.wait()
            for v in range(nbuf):
                off = off0 + v * _CHUNK
                pltpu.sync_copy(rows_v.at[v], out_hbm.at[pl.ds(off, _CHUNK)])
            return carry

        lax.fori_loop(0, n_chunks // nbuf, step, 0, unroll=False)

    call = pl.kernel(
        body,
        out_type=jax.ShapeDtypeStruct((m, width), jnp.float32),
        mesh=mesh,
        scratch_types=[
            pltpu.VMEM((nbuf, _CHUNK), jnp.int32),
            pltpu.VMEM((nbuf, _CHUNK, width), jnp.float32),
            pltpu.SemaphoreType.DMA,
        ],
    )
    return call(table, idx)


def _bf16_dot(x, w):
    return jnp.dot(x.astype(jnp.bfloat16), w.astype(jnp.bfloat16),
                   preferred_element_type=jnp.float32)


def _edge_mlp_body(f_ref, g_ref, w1_ref, b1_ref, o_ref, *, c, w2_ref=None,
                   b2_ref=None):
    center = f_ref[0]
    acc = None
    for k in range(KNN):
        nbr = g_ref[k, 0][:, :c]
        cat = jnp.concatenate([center, nbr - center], axis=1)
        h = jnp.maximum(_bf16_dot(cat, w1_ref[...]) + b1_ref[...], 0.0)
        if w2_ref is not None:
            h = jnp.maximum(_bf16_dot(h, w2_ref[...]) + b2_ref[...], 0.0)
        acc = h if acc is None else jnp.maximum(acc, h)
    o_ref[0] = acc


def _edge_tail(feat, g_t, w1, b1, w2=None, b2=None):
    bb, n, c = feat.shape
    f = w1.shape[1]
    blk = 512
    grid = (bb, n // blk)
    f_spec = pl.BlockSpec((1, blk, c), lambda b, i: (b, i, 0))
    g_spec = pl.BlockSpec((KNN, 1, blk, 128), lambda b, i: (0, b, i, 0))
    o_spec = pl.BlockSpec((1, blk, f), lambda b, i: (b, i, 0))
    out_shape = jax.ShapeDtypeStruct((bb, n, f), jnp.float32)
    w1_spec = pl.BlockSpec(w1.shape, lambda b, i: (0, 0))
    b1_spec = pl.BlockSpec((1, f), lambda b, i: (0, 0))
    if w2 is None:
        body = functools.partial(_edge_mlp_body, c=c)
        return pl.pallas_call(
            body, grid=grid,
            in_specs=[f_spec, g_spec, w1_spec, b1_spec],
            out_specs=o_spec, out_shape=out_shape,
        )(feat, g_t, w1, b1.reshape(1, f))

    def body2(f_ref, g_ref, w1_ref, b1_ref, w2_ref, b2_ref, o_ref):
        _edge_mlp_body(f_ref, g_ref, w1_ref, b1_ref, o_ref, c=c,
                       w2_ref=w2_ref, b2_ref=b2_ref)

    return pl.pallas_call(
        body2, grid=grid,
        in_specs=[f_spec, g_spec, w1_spec, b1_spec,
                  pl.BlockSpec((f, f), lambda b, i: (0, 0)),
                  pl.BlockSpec((1, f), lambda b, i: (0, 0))],
        out_specs=o_spec, out_shape=out_shape,
    )(feat, g_t, w1, b1.reshape(1, f), w2, b2.reshape(1, f))


def _glob_body(f1_ref, f2_ref, f3_ref, w_ref, b_ref, loc_ref, gmax_ref,
               gsum_ref):
    i = pl.program_id(1)
    cat = jnp.concatenate([f1_ref[0], f2_ref[0], f3_ref[0]], axis=1)
    loc = jnp.maximum(_bf16_dot(cat, w_ref[...]) + b_ref[...], 0.0)
    loc_ref[0] = loc
    pmax = jnp.max(loc, axis=0, keepdims=True)
    psum = jnp.sum(loc, axis=0, keepdims=True)

    @pl.when(i == 0)
    def _():
        gmax_ref[0] = pmax
        gsum_ref[0] = psum

    @pl.when(i != 0)
    def _():
        gmax_ref[0] = jnp.maximum(gmax_ref[0], pmax)
        gsum_ref[0] = gsum_ref[0] + psum


def _global_feats(f1, f2, f3, w41, b41):
    bb, n, f = f1.shape
    blk = 512
    co = w41.shape[1]
    f_spec = pl.BlockSpec((1, blk, f), lambda b, i: (b, i, 0))
    loc, gmax, gsum = pl.pallas_call(
        _glob_body,
        grid=(bb, n // blk),
        in_specs=[f_spec, f_spec, f_spec,
                  pl.BlockSpec((3 * f, co), lambda b, i: (0, 0)),
                  pl.BlockSpec((1, co), lambda b, i: (0, 0))],
        out_specs=[pl.BlockSpec((1, blk, co), lambda b, i: (b, i, 0)),
                   pl.BlockSpec((1, 1, co), lambda b, i: (b, 0, 0)),
                   pl.BlockSpec((1, 1, co), lambda b, i: (b, 0, 0))],
        out_shape=[jax.ShapeDtypeStruct((bb, n, co), jnp.float32),
                   jax.ShapeDtypeStruct((bb, 1, co), jnp.float32),
                   jax.ShapeDtypeStruct((bb, 1, co), jnp.float32)],
    )(f1, f2, f3, w41, b41.reshape(1, co))
    return loc, gmax.reshape(bb, co), gsum.reshape(bb, co) * (1.0 / n)


def _dense_body(x_ref, w_ref, b_ref, o_ref, *, leaky):
    h = _bf16_dot(x_ref[...], w_ref[...]) + b_ref[...]
    if leaky:
        h = jnp.where(h > 0.0, h, 0.2 * h)
    o_ref[...] = h


def _dense(x, w, bias, leaky):
    rows, cin = x.shape
    cout = w.shape[1]
    blk = 512
    return pl.pallas_call(
        functools.partial(_dense_body, leaky=leaky),
        grid=(cout // blk,),
        in_specs=[pl.BlockSpec((rows, cin), lambda j: (0, 0)),
                  pl.BlockSpec((cin, blk), lambda j: (0, j)),
                  pl.BlockSpec((1, blk), lambda j: (0, j))],
        out_specs=pl.BlockSpec((rows, blk), lambda j: (0, j)),
        out_shape=jax.ShapeDtypeStruct((rows, cout), jnp.float32),
    )(x, w, bias.reshape(1, cout))


def _edge_conv(x_knn, feat, w1, b1, w2, b2):
    bb, n, c = feat.shape
    idx = _knn_topk(x_knn)
    idx_t = jnp.swapaxes(jnp.swapaxes(idx, 0, 2), 1, 2)
    feat_pad = jnp.pad(feat, ((0, 0), (0, 0), (0, 128 - c)))
    gathered = _sc_gather(feat_pad.reshape(bb * n, 128), idx_t.reshape(-1))
    g_t = gathered.reshape(KNN, bb, n, 128)
    return _edge_tail(feat, g_t, w1, b1, w2, b2)


def _encode(x, w11, b11, w12, b12, w21, b21, w22, b22, w31, b31, w41, b41):
    x8 = jnp.pad(x, ((0, 0), (0, 0), (0, 5)))
    f1 = _edge_conv(x8, x, w11, b11, w12, b12)
    f2 = _edge_conv(f1, f1, w21, b21, w22, b22)
    f3 = _edge_conv(f2, f2, w31, b31, None, None)
    local, gmax, gavg = _global_feats(f1, f2, f3, w41, b41)
    glob = jnp.concatenate([gmax, gavg], axis=-1)
    return local, glob


def kernel(source, target, w11, b11, w12, b12, w21, b21, w22, b22, w31, b31,
           w41, b41, dw1, db1, dw2, db2, dw3, db3):
    enc_w = (w11, b11, w12, b12, w21, b21, w22, b22, w31, b31, w41, b41)
    local_x, glob_x = _encode(source, *enc_w)
    local_y, glob_y = _encode(target, *enc_w)

    glob = jnp.concatenate([glob_x, glob_y], axis=0)
    h = _dense(glob, dw1, db1, leaky=True)
    h = _dense(h, dw2, db2, leaky=True)
    rec = _dense(h, dw3, db3, leaky=False)
    rec = rec.reshape(-1, N, 3)

    half = rec.shape[0] // 2
    return ((rec[:half], rec[half:]),
            (local_x, local_y),
            (glob_x, glob_y))


# speedup vs baseline: 1.0283x; 1.0283x over previous
import functools

import jax
import jax.numpy as jnp
from jax import lax
from jax.experimental import pallas as pl
from jax.experimental.pallas import tpu as pltpu
from jax.experimental.pallas import tpu_sc as plsc

N = 2048
KNN = 20
F = 64
NEG_BIG = -1e30


def _knn_body(x_ref, xt_ref, idx_ref, *, blk, n, k):
    b = pl.program_id(0)
    xb = x_ref[0]
    xt = xt_ref[0]
    inner = jnp.dot(xb.astype(jnp.bfloat16), xt.astype(jnp.bfloat16),
                    preferred_element_type=jnp.float32)
    sq_row = jnp.sum(xt * xt, axis=0, keepdims=True)
    sq_i = jnp.sum(xb * xb, axis=1, keepdims=True)
    neg = -(sq_i - 2.0 * inner + sq_row)
    lane = lax.broadcasted_iota(jnp.int32, (blk, n), 1)
    cols = []
    for _ in range(k):
        j = jnp.argmax(neg, axis=1).astype(jnp.int32)[:, None]
        cols.append(j)
        neg = jnp.where(lane == j, NEG_BIG, neg)
    idx_ref[0] = jnp.concatenate(cols, axis=1) + b * n


def _knn_topk(x):
    bb, n, c = x.shape
    blk = 512
    xt = jnp.swapaxes(x, 1, 2)
    return pl.pallas_call(
        functools.partial(_knn_body, blk=blk, n=n, k=KNN),
        grid=(bb, n // blk),
        in_specs=[
            pl.BlockSpec((1, blk, c), lambda b, i: (b, i, 0)),
            pl.BlockSpec((1, c, n), lambda b, i: (b, 0, 0)),
        ],
        out_specs=pl.BlockSpec((1, blk, KNN), lambda b, i: (b, i, 0)),
        out_shape=jax.ShapeDtypeStruct((bb, n, KNN), jnp.int32),
    )(x, xt)


_CHUNK = 128


def _sc_gather(table, idx):
    m = idx.shape[0]
    width = table.shape[1]
    nw = 32
    rows_per_w = m // nw
    n_chunks = rows_per_w // _CHUNK
    mesh = plsc.VectorSubcoreMesh(core_axis_name="c", subcore_axis_name="s")

    def body(table_hbm, idx_hbm, out_hbm, idx_v, rows_v, sem):
        wid = lax.axis_index("s") * 2 + lax.axis_index("c")
        base = wid * rows_per_w

        def step(ci, carry):
            off = base + ci * _CHUNK
            pltpu.sync_copy(idx_hbm.at[pl.ds(off, _CHUNK)], idx_v)
            pltpu.async_copy(table_hbm.at[idx_v], rows_v, sem).wait()
            pltpu.sync_copy(rows_v, out_hbm.at[pl.ds(off, _CHUNK)])
            return carry

        lax.fori_loop(0, n_chunks, step, 0, unroll=False)

    call = pl.kernel(
        body,
        out_type=jax.ShapeDtypeStruct((m, width), jnp.float32),
        mesh=mesh,
        scratch_types=[
            pltpu.VMEM((_CHUNK,), jnp.int32),
            pltpu.VMEM((_CHUNK, width), jnp.float32),
            pltpu.SemaphoreType.DMA,
        ],
    )
    return call(table, idx)


def _bf16_dot(x, w):
    return jnp.dot(x.astype(jnp.bfloat16), w.astype(jnp.bfloat16),
                   preferred_element_type=jnp.float32)


def _edge_mlp_body(f_ref, g_ref, w1_ref, b1_ref, o_ref, *, c, w2_ref=None,
                   b2_ref=None):
    center = f_ref[0]
    acc = None
    for k in range(KNN):
        nbr = g_ref[k, 0][:, :c]
        cat = jnp.concatenate([center, nbr - center], axis=1)
        h = jnp.maximum(_bf16_dot(cat, w1_ref[...]) + b1_ref[...], 0.0)
        if w2_ref is not None:
            h = jnp.maximum(_bf16_dot(h, w2_ref[...]) + b2_ref[...], 0.0)
        acc = h if acc is None else jnp.maximum(acc, h)
    o_ref[0] = acc


def _edge_tail(feat, g_t, w1, b1, w2=None, b2=None):
    bb, n, c = feat.shape
    f = w1.shape[1]
    blk = 512
    grid = (bb, n // blk)
    f_spec = pl.BlockSpec((1, blk, c), lambda b, i: (b, i, 0))
    g_spec = pl.BlockSpec((KNN, 1, blk, 128), lambda b, i: (0, b, i, 0))
    o_spec = pl.BlockSpec((1, blk, f), lambda b, i: (b, i, 0))
    out_shape = jax.ShapeDtypeStruct((bb, n, f), jnp.float32)
    w1_spec = pl.BlockSpec(w1.shape, lambda b, i: (0, 0))
    b1_spec = pl.BlockSpec((1, f), lambda b, i: (0, 0))
    if w2 is None:
        body = functools.partial(_edge_mlp_body, c=c)
        return pl.pallas_call(
            body, grid=grid,
            in_specs=[f_spec, g_spec, w1_spec, b1_spec],
            out_specs=o_spec, out_shape=out_shape,
        )(feat, g_t, w1, b1.reshape(1, f))

    def body2(f_ref, g_ref, w1_ref, b1_ref, w2_ref, b2_ref, o_ref):
        _edge_mlp_body(f_ref, g_ref, w1_ref, b1_ref, o_ref, c=c,
                       w2_ref=w2_ref, b2_ref=b2_ref)

    return pl.pallas_call(
        body2, grid=grid,
        in_specs=[f_spec, g_spec, w1_spec, b1_spec,
                  pl.BlockSpec((f, f), lambda b, i: (0, 0)),
                  pl.BlockSpec((1, f), lambda b, i: (0, 0))],
        out_specs=o_spec, out_shape=out_shape,
    )(feat, g_t, w1, b1.reshape(1, f), w2, b2.reshape(1, f))


def _glob_body(f1_ref, f2_ref, f3_ref, w_ref, b_ref, loc_ref, gmax_ref,
               gsum_ref):
    i = pl.program_id(1)
    cat = jnp.concatenate([f1_ref[0], f2_ref[0], f3_ref[0]], axis=1)
    loc = jnp.maximum(_bf16_dot(cat, w_ref[...]) + b_ref[...], 0.0)
    loc_ref[0] = loc
    pmax = jnp.max(loc, axis=0, keepdims=True)
    psum = jnp.sum(loc, axis=0, keepdims=True)

    @pl.when(i == 0)
    def _():
        gmax_ref[0] = pmax
        gsum_ref[0] = psum

    @pl.when(i != 0)
    def _():
        gmax_ref[0] = jnp.maximum(gmax_ref[0], pmax)
        gsum_ref[0] = gsum_ref[0] + psum


def _global_feats(f1, f2, f3, w41, b41):
    bb, n, f = f1.shape
    blk = 512
    co = w41.shape[1]
    f_spec = pl.BlockSpec((1, blk, f), lambda b, i: (b, i, 0))
    loc, gmax, gsum = pl.pallas_call(
        _glob_body,
        grid=(bb, n // blk),
        in_specs=[f_spec, f_spec, f_spec,
                  pl.BlockSpec((3 * f, co), lambda b, i: (0, 0)),
                  pl.BlockSpec((1, co), lambda b, i: (0, 0))],
        out_specs=[pl.BlockSpec((1, blk, co), lambda b, i: (b, i, 0)),
                   pl.BlockSpec((1, 1, co), lambda b, i: (b, 0, 0)),
                   pl.BlockSpec((1, 1, co), lambda b, i: (b, 0, 0))],
        out_shape=[jax.ShapeDtypeStruct((bb, n, co), jnp.float32),
                   jax.ShapeDtypeStruct((bb, 1, co), jnp.float32),
                   jax.ShapeDtypeStruct((bb, 1, co), jnp.float32)],
    )(f1, f2, f3, w41, b41.reshape(1, co))
    return loc, gmax.reshape(bb, co), gsum.reshape(bb, co) * (1.0 / n)


def _dense_body(x_ref, w_ref, b_ref, o_ref, *, leaky):
    h = _bf16_dot(x_ref[...], w_ref[...]) + b_ref[...]
    if leaky:
        h = jnp.where(h > 0.0, h, 0.2 * h)
    o_ref[...] = h


def _dense(x, w, bias, leaky):
    rows, cin = x.shape
    cout = w.shape[1]
    blk = 512
    return pl.pallas_call(
        functools.partial(_dense_body, leaky=leaky),
        grid=(cout // blk,),
        in_specs=[pl.BlockSpec((rows, cin), lambda j: (0, 0)),
                  pl.BlockSpec((cin, blk), lambda j: (0, j)),
                  pl.BlockSpec((1, blk), lambda j: (0, j))],
        out_specs=pl.BlockSpec((rows, blk), lambda j: (0, j)),
        out_shape=jax.ShapeDtypeStruct((rows, cout), jnp.float32),
    )(x, w, bias.reshape(1, cout))


def _edge_conv(x_knn, feat, w1, b1, w2, b2):
    bb, n, c = feat.shape
    idx = _knn_topk(x_knn)
    idx_t = jnp.swapaxes(jnp.swapaxes(idx, 0, 2), 1, 2)
    feat_pad = jnp.pad(feat, ((0, 0), (0, 0), (0, 128 - c)))
    gathered = _sc_gather(feat_pad.reshape(bb * n, 128), idx_t.reshape(-1))
    g_t = gathered.reshape(KNN, bb, n, 128)
    return _edge_tail(feat, g_t, w1, b1, w2, b2)


def _encode(x, w11, b11, w12, b12, w21, b21, w22, b22, w31, b31, w41, b41):
    x8 = jnp.pad(x, ((0, 0), (0, 0), (0, 5)))
    f1 = _edge_conv(x8, x, w11, b11, w12, b12)
    f2 = _edge_conv(f1, f1, w21, b21, w22, b22)
    f3 = _edge_conv(f2, f2, w31, b31, None, None)
    local, gmax, gavg = _global_feats(f1, f2, f3, w41, b41)
    glob = jnp.concatenate([gmax, gavg], axis=-1)
    return local, glob


def kernel(source, target, w11, b11, w12, b12, w21, b21, w22, b22, w31, b31,
           w41, b41, dw1, db1, dw2, db2, dw3, db3):
    enc_w = (w11, b11, w12, b12, w21, b21, w22, b22, w31, b31, w41, b41)
    local_x, glob_x = _encode(source, *enc_w)
    local_y, glob_y = _encode(target, *enc_w)

    glob = jnp.concatenate([glob_x, glob_y], axis=0)
    h = _dense(glob, dw1, db1, leaky=True)
    h = _dense(h, dw2, db2, leaky=True)
    rec = _dense(h, dw3, db3, leaky=False)
    rec = rec.reshape(-1, N, 3)

    half = rec.shape[0] // 2
    return ((rec[:half], rec[half:]),
            (local_x, local_y),
            (glob_x, glob_y))
